# baseline (speedup 1.0000x reference)
import functools

import jax
import jax.numpy as jnp
from jax import lax
from jax.experimental import pallas as pl
from jax.experimental.pallas import tpu as pltpu
from jax.experimental.pallas import tpu_sc as plsc

N_NODES = 50000
D_FEAT = 256
NUM_CLUSTERS = 12500

NSC = 2
NT = 16
LANES = 16
HD = D_FEAT // NSC
HV = HD // LANES
SL = 8
NRT = N_NODES // SL
RPT = 3128
RPT_LAST = N_NODES - (NT - 1) * RPT
SUB = 64
NSUB = 48
TAIL = RPT - NSUB * SUB
TAIL_L = RPT_LAST - NSUB * SUB
CW = 16
CPAD = 12544
CPT = CPAD // NT
NZ = CPT // SUB
ZTAIL = CPT - NZ * SUB


def _sc_body(x_hbm, cm_hbm, sums_hbm, cnts_hbm,
             acc, cacc, ids, xba, xbb, ones, sxa, sxb, semc, sas, sbs):
    c = lax.axis_index("c")
    s = lax.axis_index("s")
    base = s * RPT
    base_rt = s * (RPT // SL)

    zeros16 = jnp.zeros((LANES,), jnp.float32)
    czero2 = jnp.zeros((2, CW), jnp.int16)
    cone2 = jnp.ones((2, CW), jnp.int16)

    def zero_bufs(r, _):
        def zv(v, _):
            xba[r, pl.ds(v * LANES, LANES)] = zeros16
            return 0
        lax.fori_loop(0, HV, zv, 0)
        ones[pl.ds(2 * r, 2), :] = czero2
        return 0
    lax.fori_loop(0, SUB, zero_bufs, 0)

    def zero_acc(k, _):
        b = s * CPT + k * SUB
        pltpu.async_copy(xba, acc.at[pl.ds(b, SUB)], sas)
        pltpu.async_copy(ones, cacc.at[pl.ds(b, SUB)], sbs)
        return 0
    lax.fori_loop(0, NZ, zero_acc, 0)
    tb = s * CPT + NZ * SUB
    pltpu.async_copy(xba.at[pl.ds(0, ZTAIL)], acc.at[pl.ds(tb, ZTAIL)], sas)
    pltpu.async_copy(ones.at[pl.ds(0, ZTAIL)], cacc.at[pl.ds(tb, ZTAIL)],
                     sbs)

    @pl.when(s < NT - 1)
    def _():
        pltpu.sync_copy(cm_hbm.at[pl.ds(base, RPT)], ids)

    @pl.when(s == NT - 1)
    def _():
        pltpu.sync_copy(cm_hbm.at[pl.ds(base, RPT_LAST)],
                        ids.at[pl.ds(0, RPT_LAST)])

    def drain_zero(k, _):
        pltpu.make_async_copy(xba, acc.at[pl.ds(tb, SUB)], sas).wait()
        pltpu.make_async_copy(ones, cacc.at[pl.ds(tb, SUB)], sbs).wait()
        return 0
    lax.fori_loop(0, NZ, drain_zero, 0)
    pltpu.make_async_copy(xba.at[pl.ds(0, ZTAIL)], acc.at[pl.ds(tb, ZTAIL)],
                          sas).wait()
    pltpu.make_async_copy(ones.at[pl.ds(0, ZTAIL)], cacc.at[pl.ds(tb, ZTAIL)],
                          sbs).wait()

    def fill_ones(i, _):
        ones[pl.ds(2 * i, 2), :] = cone2
        return 0
    lax.fori_loop(0, SUB // 2, fill_ones, 0)

    plsc.subcore_barrier()

    def load_sub(k, buf, sem):
        def ld(m, _):
            pltpu.async_copy(x_hbm.at[base_rt + k * SL + m, c],
                             buf.at[pl.ds(SL * m, SL)], sem)
            return 0
        lax.fori_loop(0, SL, ld, 0)

    def wait_sub(buf, sem):
        def wt(m, _):
            pltpu.make_async_copy(x_hbm.at[base_rt, c],
                                  buf.at[pl.ds(SL * m, SL)], sem).wait()
            return 0
        lax.fori_loop(0, SL, wt, 0)

    load_sub(0, xba, sxa)
    load_sub(1, xbb, sxb)

    def accum(j, _):
        ka = 2 * j
        kb = 2 * j + 1
        wait_sub(xba, sxa)
        pltpu.async_copy(xba, acc.at[ids.at[pl.ds(ka * SUB, SUB)]], sas,
                         add=True)
        pltpu.async_copy(ones, cacc.at[ids.at[pl.ds(ka * SUB, SUB)]],
                         semc, add=True)
        wait_sub(xbb, sxb)
        pltpu.async_copy(xbb, acc.at[ids.at[pl.ds(kb * SUB, SUB)]], sbs,
                         add=True)
        pltpu.async_copy(ones, cacc.at[ids.at[pl.ds(kb * SUB, SUB)]],
                         semc, add=True)
        pltpu.make_async_copy(xba, acc.at[ids.at[pl.ds(0, SUB)]], sas).wait()
        load_sub(jnp.minimum(ka + 2, NSUB - 1), xba, sxa)
        pltpu.make_async_copy(xbb, acc.at[ids.at[pl.ds(0, SUB)]], sbs).wait()
        load_sub(jnp.minimum(kb + 2, NSUB - 1), xbb, sxb)
        return 0
    lax.fori_loop(0, NSUB // 2, accum, 0)
    wait_sub(xba, sxa)
    wait_sub(xbb, sxb)

    def drain_counts(k, _):
        pltpu.make_async_copy(ones, cacc.at[ids.at[pl.ds(0, SUB)]],
                              semc).wait()
        return 0
    lax.fori_loop(0, NSUB, drain_counts, 0)

    trt = base_rt + NSUB * SL

    @pl.when(s < NT - 1)
    def _():
        def ld(m, _):
            pltpu.async_copy(x_hbm.at[trt + m, c],
                             xba.at[pl.ds(SL * m, SL)], sxa)
            return 0
        lax.fori_loop(0, TAIL // SL, ld, 0)

        def wt(m, _):
            pltpu.make_async_copy(x_hbm.at[trt, c],
                                  xba.at[pl.ds(SL * m, SL)], sxa).wait()
            return 0
        lax.fori_loop(0, TAIL // SL, wt, 0)
        isl = ids.at[pl.ds(NSUB * SUB, TAIL)]
        pltpu.sync_copy(xba.at[pl.ds(0, TAIL)], acc.at[isl], add=True)
        pltpu.sync_copy(ones.at[pl.ds(0, TAIL)], cacc.at[isl], add=True)

    @pl.when(s == NT - 1)
    def _():
        pltpu.sync_copy(x_hbm.at[trt, c], xba.at[pl.ds(0, TAIL_L)])
        isl = ids.at[pl.ds(NSUB * SUB, TAIL_L)]
        pltpu.sync_copy(xba.at[pl.ds(0, TAIL_L)], acc.at[isl], add=True)
        pltpu.sync_copy(ones.at[pl.ds(0, TAIL_L)], cacc.at[isl], add=True)

    plsc.subcore_barrier()

    sl = pl.ds(s * CPT, CPT)
    pltpu.sync_copy(acc.at[sl], sums_hbm.at[c, sl])

    @pl.when(c == 0)
    def _():
        pltpu.sync_copy(cacc.at[sl], cnts_hbm.at[sl])


def _tc_divide(sums_ref, cnts_ref, out_ref):
    cnt = cnts_ref[:, 0:1].astype(jnp.int32) & 0xFFFF
    inv = 1.0 / jnp.maximum(cnt.astype(jnp.float32), 1.0)
    out_ref[:, :HD] = sums_ref[0] * inv
    out_ref[:, HD:] = sums_ref[1] * inv


@jax.jit
def _pooled(x4, cm):
    mesh = plsc.VectorSubcoreMesh(core_axis_name="c", subcore_axis_name="s")
    f = functools.partial(
        pl.kernel,
        mesh=mesh,
        out_type=(
            jax.ShapeDtypeStruct((NSC, CPAD, HD), jnp.float32),
            jax.ShapeDtypeStruct((CPAD, CW), jnp.int16),
        ),
        scratch_types=[
            pltpu.VMEM_SHARED((CPAD, HD), jnp.float32),
            pltpu.VMEM_SHARED((CPAD, CW), jnp.int16),
            pltpu.VMEM((RPT,), jnp.int32),
            pltpu.VMEM((SUB, HD), jnp.float32),
            pltpu.VMEM((SUB, HD), jnp.float32),
            pltpu.VMEM((SUB, CW), jnp.int16),
            pltpu.SemaphoreType.DMA,
            pltpu.SemaphoreType.DMA,
            pltpu.SemaphoreType.DMA,
            pltpu.SemaphoreType.DMA,
            pltpu.SemaphoreType.DMA,
        ],
        compiler_params=pltpu.CompilerParams(
            use_tc_tiling_on_sc=False, needs_layout_passes=False
        ),
    )(_sc_body)
    sums, cnts = f(x4, cm)

    blk = 2 * CPT
    out = pl.pallas_call(
        _tc_divide,
        grid=(CPAD // blk,),
        in_specs=[
            pl.BlockSpec((NSC, blk, HD), lambda i: (0, i, 0)),
            pl.BlockSpec((blk, CW), lambda i: (i, 0)),
        ],
        out_specs=pl.BlockSpec((blk, D_FEAT), lambda i: (i, 0)),
        out_shape=jax.ShapeDtypeStruct((NUM_CLUSTERS, D_FEAT), jnp.float32),
    )(sums, cnts)
    return out


def kernel(x, cluster_map, edge_index):
    x4 = x.reshape(NRT, SL, NSC, HD).transpose(0, 2, 1, 3)
    return _pooled(x4, cluster_map), edge_index


# speedup vs baseline: 1.0522x; 1.0522x over previous
import functools

import jax
import jax.numpy as jnp
from jax import lax
from jax.experimental import pallas as pl
from jax.experimental.pallas import tpu as pltpu
from jax.experimental.pallas import tpu_sc as plsc

N_NODES = 50000
D_FEAT = 256
NUM_CLUSTERS = 12500

NSC = 2
NT = 16
LANES = 16
HD = D_FEAT // NSC
HV = HD // LANES
SL = 8
NRT = N_NODES // SL
RPT = 3128
RPT_LAST = N_NODES - (NT - 1) * RPT
SUB = 64
NSUB = 48
TAIL = RPT - NSUB * SUB
TAIL_L = RPT_LAST - NSUB * SUB
CW = 16
CPAD = 12544
CPT = CPAD // NT
NZ = CPT // SUB
ZTAIL = CPT - NZ * SUB


def _sc_body(x_hbm, cm_hbm, sums_hbm, cnts_hbm,
             acc, cacc, ids, xba, xbb, ones, sxa, sxb, semc):
    c = lax.axis_index("c")
    s = lax.axis_index("s")
    base = s * RPT
    base_rt = s * (RPT // SL)

    zeros16 = jnp.zeros((LANES,), jnp.float32)
    czero2 = jnp.zeros((2, CW), jnp.int16)
    cone2 = jnp.ones((2, CW), jnp.int16)

    def zero_bufs(r, _):
        def zv(v, _):
            xba[r, pl.ds(v * LANES, LANES)] = zeros16
            return 0
        lax.fori_loop(0, HV, zv, 0)
        ones[pl.ds(2 * r, 2), :] = czero2
        return 0
    lax.fori_loop(0, SUB, zero_bufs, 0)

    def zero_acc(k, _):
        b = s * CPT + k * SUB
        pltpu.sync_copy(xba, acc.at[pl.ds(b, SUB)])
        pltpu.sync_copy(ones, cacc.at[pl.ds(b, SUB)])
        return 0
    lax.fori_loop(0, NZ, zero_acc, 0)
    tb = s * CPT + NZ * SUB
    pltpu.sync_copy(xba.at[pl.ds(0, ZTAIL)], acc.at[pl.ds(tb, ZTAIL)])
    pltpu.sync_copy(ones.at[pl.ds(0, ZTAIL)], cacc.at[pl.ds(tb, ZTAIL)])

    def fill_ones(i, _):
        ones[pl.ds(2 * i, 2), :] = cone2
        return 0
    lax.fori_loop(0, SUB // 2, fill_ones, 0)

    @pl.when(s < NT - 1)
    def _():
        pltpu.sync_copy(cm_hbm.at[pl.ds(base, RPT)], ids)

    @pl.when(s == NT - 1)
    def _():
        pltpu.sync_copy(cm_hbm.at[pl.ds(base, RPT_LAST)],
                        ids.at[pl.ds(0, RPT_LAST)])

    plsc.subcore_barrier()

    def load_sub(k, buf, sem):
        def ld(m, _):
            pltpu.async_copy(x_hbm.at[base_rt + k * SL + m, c],
                             buf.at[pl.ds(SL * m, SL)], sem)
            return 0
        lax.fori_loop(0, SL, ld, 0)

    def wait_sub(buf, sem):
        def wt(m, _):
            pltpu.make_async_copy(x_hbm.at[base_rt, c],
                                  buf.at[pl.ds(SL * m, SL)], sem).wait()
            return 0
        lax.fori_loop(0, SL, wt, 0)

    load_sub(0, xba, sxa)

    def accum(j, _):
        ka = 2 * j
        kb = 2 * j + 1
        load_sub(kb, xbb, sxb)
        wait_sub(xba, sxa)
        pltpu.sync_copy(xba, acc.at[ids.at[pl.ds(ka * SUB, SUB)]], add=True)
        pltpu.async_copy(ones, cacc.at[ids.at[pl.ds(ka * SUB, SUB)]],
                         semc, add=True)
        kn = jnp.minimum(ka + 2, NSUB - 1)
        load_sub(kn, xba, sxa)
        wait_sub(xbb, sxb)
        pltpu.sync_copy(xbb, acc.at[ids.at[pl.ds(kb * SUB, SUB)]], add=True)
        pltpu.async_copy(ones, cacc.at[ids.at[pl.ds(kb * SUB, SUB)]],
                         semc, add=True)
        return 0
    lax.fori_loop(0, NSUB // 2, accum, 0)
    wait_sub(xba, sxa)

    def drain_counts(k, _):
        pltpu.make_async_copy(ones, cacc.at[ids.at[pl.ds(0, SUB)]],
                              semc).wait()
        return 0
    lax.fori_loop(0, NSUB, drain_counts, 0)

    trt = base_rt + NSUB * SL

    @pl.when(s < NT - 1)
    def _():
        def ld(m, _):
            pltpu.async_copy(x_hbm.at[trt + m, c],
                             xba.at[pl.ds(SL * m, SL)], sxa)
            return 0
        lax.fori_loop(0, TAIL // SL, ld, 0)

        def wt(m, _):
            pltpu.make_async_copy(x_hbm.at[trt, c],
                                  xba.at[pl.ds(SL * m, SL)], sxa).wait()
            return 0
        lax.fori_loop(0, TAIL // SL, wt, 0)
        isl = ids.at[pl.ds(NSUB * SUB, TAIL)]
        pltpu.sync_copy(xba.at[pl.ds(0, TAIL)], acc.at[isl], add=True)
        pltpu.sync_copy(ones.at[pl.ds(0, TAIL)], cacc.at[isl], add=True)

    @pl.when(s == NT - 1)
    def _():
        pltpu.sync_copy(x_hbm.at[trt, c], xba.at[pl.ds(0, TAIL_L)])
        isl = ids.at[pl.ds(NSUB * SUB, TAIL_L)]
        pltpu.sync_copy(xba.at[pl.ds(0, TAIL_L)], acc.at[isl], add=True)
        pltpu.sync_copy(ones.at[pl.ds(0, TAIL_L)], cacc.at[isl], add=True)

    plsc.subcore_barrier()

    sl = pl.ds(s * CPT, CPT)
    pltpu.sync_copy(acc.at[sl], sums_hbm.at[c, sl])

    @pl.when(c == 0)
    def _():
        pltpu.sync_copy(cacc.at[sl], cnts_hbm.at[sl])


def _tc_divide(sums_ref, cnts_ref, out_ref):
    cnt = cnts_ref[:, 0:1].astype(jnp.int32) & 0xFFFF
    inv = 1.0 / jnp.maximum(cnt.astype(jnp.float32), 1.0)
    nrow = cnts_ref.shape[0]
    out_ref[:, :HD] = sums_ref[0].reshape(nrow, HD) * inv
    out_ref[:, HD:] = sums_ref[1].reshape(nrow, HD) * inv


@jax.jit
def _pooled(x4, cm):
    mesh = plsc.VectorSubcoreMesh(core_axis_name="c", subcore_axis_name="s")
    f = functools.partial(
        pl.kernel,
        mesh=mesh,
        out_type=(
            jax.ShapeDtypeStruct((NSC, CPAD, HD), jnp.float32),
            jax.ShapeDtypeStruct((CPAD, CW), jnp.int16),
        ),
        scratch_types=[
            pltpu.VMEM_SHARED((CPAD, HD), jnp.float32),
            pltpu.VMEM_SHARED((CPAD, CW), jnp.int16),
            pltpu.VMEM((RPT,), jnp.int32),
            pltpu.VMEM((SUB, HD), jnp.float32),
            pltpu.VMEM((SUB, HD), jnp.float32),
            pltpu.VMEM((SUB, CW), jnp.int16),
            pltpu.SemaphoreType.DMA,
            pltpu.SemaphoreType.DMA,
            pltpu.SemaphoreType.DMA,
        ],
        compiler_params=pltpu.CompilerParams(
            use_tc_tiling_on_sc=False, needs_layout_passes=False
        ),
    )(_sc_body)
    sums, cnts = f(x4, cm)
    sums4 = sums.reshape(NSC, CPAD // SL, SL, HD)

    blk = 2 * CPT
    out = pl.pallas_call(
        _tc_divide,
        grid=(CPAD // blk,),
        in_specs=[
            pl.BlockSpec((NSC, blk // SL, SL, HD), lambda i: (0, i, 0, 0)),
            pl.BlockSpec((blk, CW), lambda i: (i, 0)),
        ],
        out_specs=pl.BlockSpec((blk, D_FEAT), lambda i: (i, 0)),
        out_shape=jax.ShapeDtypeStruct((NUM_CLUSTERS, D_FEAT), jnp.float32),
    )(sums4, cnts)
    return out


def kernel(x, cluster_map, edge_index):
    x4 = x.reshape(NRT, SL, NSC, HD).transpose(0, 2, 1, 3)
    return _pooled(x4, cluster_map), edge_index


# speedup vs baseline: 1.0664x; 1.0135x over previous
import functools

import jax
import jax.numpy as jnp
from jax import lax
from jax.experimental import pallas as pl
from jax.experimental.pallas import tpu as pltpu
from jax.experimental.pallas import tpu_sc as plsc

N_NODES = 50000
D_FEAT = 256
NUM_CLUSTERS = 12500

NSC = 2
NT = 16
LANES = 16
HD = D_FEAT // NSC
HV = HD // LANES
SL = 8
NRT = N_NODES // SL
RPT = 3128
RPT_LAST = N_NODES - (NT - 1) * RPT
SUB = 64
NSUB = 48
TAIL = RPT - NSUB * SUB
TAIL_L = RPT_LAST - NSUB * SUB
CW = 16
CPAD = 12544
CPT = CPAD // NT
NZ = CPT // SUB
ZTAIL = CPT - NZ * SUB


def _sc_body(x_hbm, cm_hbm, sums_hbm, cnts_hbm,
             acc, cacc, ids, xba, xbb, ones, sxa, sxb, semc):
    c = lax.axis_index("c")
    s = lax.axis_index("s")
    base = s * RPT
    base_rt = s * (RPT // SL)

    zeros16 = jnp.zeros((LANES,), jnp.float32)
    czero2 = jnp.zeros((2, CW), jnp.int16)
    cone2 = jnp.ones((2, CW), jnp.int16)

    def zero_bufs(r, _):
        def zv(v, _):
            xba[r, pl.ds(v * LANES, LANES)] = zeros16
            return 0
        lax.fori_loop(0, HV, zv, 0)
        ones[pl.ds(2 * r, 2), :] = czero2
        return 0
    lax.fori_loop(0, SUB, zero_bufs, 0)

    def zero_acc(k, _):
        b = s * CPT + k * SUB
        pltpu.sync_copy(xba, acc.at[pl.ds(b, SUB)])
        pltpu.sync_copy(ones, cacc.at[pl.ds(b, SUB)])
        return 0
    lax.fori_loop(0, NZ, zero_acc, 0)
    tb = s * CPT + NZ * SUB
    pltpu.sync_copy(xba.at[pl.ds(0, ZTAIL)], acc.at[pl.ds(tb, ZTAIL)])
    pltpu.sync_copy(ones.at[pl.ds(0, ZTAIL)], cacc.at[pl.ds(tb, ZTAIL)])

    def fill_ones(i, _):
        ones[pl.ds(2 * i, 2), :] = cone2
        return 0
    lax.fori_loop(0, SUB // 2, fill_ones, 0)

    @pl.when(s < NT - 1)
    def _():
        pltpu.sync_copy(cm_hbm.at[pl.ds(base, RPT)], ids)

    @pl.when(s == NT - 1)
    def _():
        pltpu.sync_copy(cm_hbm.at[pl.ds(base, RPT_LAST)],
                        ids.at[pl.ds(0, RPT_LAST)])

    plsc.subcore_barrier()

    def load_sub(k, buf, sem):
        def ld(m, _):
            pltpu.async_copy(x_hbm.at[base_rt + k * SL + m, c],
                             buf.at[pl.ds(SL * m, SL)], sem)
            return 0
        lax.fori_loop(0, SL, ld, 0)

    def wait_sub(buf, sem):
        def wt(m, _):
            pltpu.make_async_copy(x_hbm.at[base_rt, c],
                                  buf.at[pl.ds(SL * m, SL)], sem).wait()
            return 0
        lax.fori_loop(0, SL, wt, 0)

    load_sub(0, xba, sxa)

    def accum(j, _):
        ka = 2 * j
        kb = 2 * j + 1
        load_sub(kb, xbb, sxb)
        wait_sub(xba, sxa)
        pltpu.sync_copy(xba, acc.at[ids.at[pl.ds(ka * SUB, SUB)]], add=True)
        pltpu.async_copy(ones, cacc.at[ids.at[pl.ds(ka * SUB, SUB)]],
                         semc, add=True)
        kn = jnp.minimum(ka + 2, NSUB - 1)
        load_sub(kn, xba, sxa)
        wait_sub(xbb, sxb)
        pltpu.sync_copy(xbb, acc.at[ids.at[pl.ds(kb * SUB, SUB)]], add=True)
        pltpu.async_copy(ones, cacc.at[ids.at[pl.ds(kb * SUB, SUB)]],
                         semc, add=True)
        return 0
    lax.fori_loop(0, NSUB // 2, accum, 0)
    wait_sub(xba, sxa)

    def drain_counts(k, _):
        pltpu.make_async_copy(ones, cacc.at[ids.at[pl.ds(0, SUB)]],
                              semc).wait()
        return 0
    lax.fori_loop(0, NSUB, drain_counts, 0)

    trt = base_rt + NSUB * SL

    @pl.when(s < NT - 1)
    def _():
        def ld(m, _):
            pltpu.async_copy(x_hbm.at[trt + m, c],
                             xba.at[pl.ds(SL * m, SL)], sxa)
            return 0
        lax.fori_loop(0, TAIL // SL, ld, 0)

        def wt(m, _):
            pltpu.make_async_copy(x_hbm.at[trt, c],
                                  xba.at[pl.ds(SL * m, SL)], sxa).wait()
            return 0
        lax.fori_loop(0, TAIL // SL, wt, 0)
        isl = ids.at[pl.ds(NSUB * SUB, TAIL)]
        pltpu.sync_copy(xba.at[pl.ds(0, TAIL)], acc.at[isl], add=True)
        pltpu.sync_copy(ones.at[pl.ds(0, TAIL)], cacc.at[isl], add=True)

    @pl.when(s == NT - 1)
    def _():
        pltpu.sync_copy(x_hbm.at[trt, c], xba.at[pl.ds(0, TAIL_L)])
        isl = ids.at[pl.ds(NSUB * SUB, TAIL_L)]
        pltpu.sync_copy(xba.at[pl.ds(0, TAIL_L)], acc.at[isl], add=True)
        pltpu.sync_copy(ones.at[pl.ds(0, TAIL_L)], cacc.at[isl], add=True)

    plsc.subcore_barrier()

    sl = pl.ds(s * CPT, CPT)
    pltpu.sync_copy(acc.at[sl], sums_hbm.at[c, sl])

    @pl.when(c == 0)
    def _():
        pltpu.sync_copy(cacc.at[sl], cnts_hbm.at[sl])


def _tc_divide(sums_ref, cnts_ref, out_ref):
    cnt = cnts_ref[:, 0:1].astype(jnp.int32) & 0xFFFF
    inv = 1.0 / jnp.maximum(cnt.astype(jnp.float32), 1.0)
    out_ref[:, :HD] = sums_ref[0] * inv
    out_ref[:, HD:] = sums_ref[1] * inv


@jax.jit
def _pooled(x4, cm):
    mesh = plsc.VectorSubcoreMesh(core_axis_name="c", subcore_axis_name="s")
    f = functools.partial(
        pl.kernel,
        mesh=mesh,
        out_type=(
            jax.ShapeDtypeStruct((NSC, CPAD, HD), jnp.float32),
            jax.ShapeDtypeStruct((CPAD, CW), jnp.int16),
        ),
        scratch_types=[
            pltpu.VMEM_SHARED((CPAD, HD), jnp.float32),
            pltpu.VMEM_SHARED((CPAD, CW), jnp.int16),
            pltpu.VMEM((RPT,), jnp.int32),
            pltpu.VMEM((SUB, HD), jnp.float32),
            pltpu.VMEM((SUB, HD), jnp.float32),
            pltpu.VMEM((SUB, CW), jnp.int16),
            pltpu.SemaphoreType.DMA,
            pltpu.SemaphoreType.DMA,
            pltpu.SemaphoreType.DMA,
        ],
        compiler_params=pltpu.CompilerParams(
            use_tc_tiling_on_sc=False, needs_layout_passes=False
        ),
    )(_sc_body)
    sums, cnts = f(x4, cm)

    blk = 4 * CPT
    out = pl.pallas_call(
        _tc_divide,
        grid=(CPAD // blk,),
        in_specs=[
            pl.BlockSpec((NSC, blk, HD), lambda i: (0, i, 0)),
            pl.BlockSpec((blk, CW), lambda i: (i, 0)),
        ],
        out_specs=pl.BlockSpec((blk, D_FEAT), lambda i: (i, 0)),
        out_shape=jax.ShapeDtypeStruct((NUM_CLUSTERS, D_FEAT), jnp.float32),
    )(sums, cnts)
    return out


def kernel(x, cluster_map, edge_index):
    x4 = x.reshape(NRT, SL, NSC, HD).transpose(0, 2, 1, 3)
    return _pooled(x4, cluster_map), edge_index


# speedup vs baseline: 1.0880x; 1.0202x over previous
import functools

import jax
import jax.numpy as jnp
from jax import lax
from jax.experimental import pallas as pl
from jax.experimental.pallas import tpu as pltpu
from jax.experimental.pallas import tpu_sc as plsc

N_NODES = 50000
D_FEAT = 256
NUM_CLUSTERS = 12500

NSC = 2
NT = 16
LANES = 16
HD = D_FEAT // NSC
HV = HD // LANES
SL = 8
NRT = N_NODES // SL
RPT = 3128
RPT_LAST = N_NODES - (NT - 1) * RPT
SUB = 64
NSUB = 48
TAIL = RPT - NSUB * SUB
TAIL_L = RPT_LAST - NSUB * SUB
CW = 16
CPAD = 12544
CPT = CPAD // NT
NZ = CPT // SUB
ZTAIL = CPT - NZ * SUB


def _sc_body(x_hbm, cm_hbm, sums_hbm, cnts_hbm,
             acc, cacc, ids, xba, xbb, ones, sxa, sxb, semc):
    c = lax.axis_index("c")
    s = lax.axis_index("s")
    base = s * RPT
    base_rt = s * (RPT // SL)

    zeros16 = jnp.zeros((LANES,), jnp.float32)
    czero2 = jnp.zeros((2, CW), jnp.int16)
    cone2 = jnp.ones((2, CW), jnp.int16)

    def zero_bufs(r, _):
        def zv(v, _):
            xba[r, pl.ds(v * LANES, LANES)] = zeros16
            return 0
        lax.fori_loop(0, HV, zv, 0)
        ones[pl.ds(2 * r, 2), :] = czero2
        return 0
    lax.fori_loop(0, SUB, zero_bufs, 0)

    def zero_acc(k, _):
        b = s * CPT + k * SUB
        pltpu.sync_copy(xba, acc.at[pl.ds(b, SUB)])
        pltpu.sync_copy(ones, cacc.at[pl.ds(b, SUB)])
        return 0
    lax.fori_loop(0, NZ, zero_acc, 0)
    tb = s * CPT + NZ * SUB
    pltpu.sync_copy(xba.at[pl.ds(0, ZTAIL)], acc.at[pl.ds(tb, ZTAIL)])
    pltpu.sync_copy(ones.at[pl.ds(0, ZTAIL)], cacc.at[pl.ds(tb, ZTAIL)])

    def fill_ones(i, _):
        ones[pl.ds(2 * i, 2), :] = cone2
        return 0
    lax.fori_loop(0, SUB // 2, fill_ones, 0)

    @pl.when(s < NT - 1)
    def _():
        pltpu.sync_copy(cm_hbm.at[pl.ds(base, RPT)], ids)

    @pl.when(s == NT - 1)
    def _():
        pltpu.sync_copy(cm_hbm.at[pl.ds(base, RPT_LAST)],
                        ids.at[pl.ds(0, RPT_LAST)])

    plsc.subcore_barrier()

    def load_sub(k, buf, sem):
        def ld(m, _):
            pltpu.async_copy(x_hbm.at[base_rt + k * SL + m, c],
                             buf.at[pl.ds(SL * m, SL)], sem)
            return 0
        lax.fori_loop(0, SL, ld, 0)

    def wait_sub(buf, sem):
        def wt(m, _):
            pltpu.make_async_copy(x_hbm.at[base_rt, c],
                                  buf.at[pl.ds(SL * m, SL)], sem).wait()
            return 0
        lax.fori_loop(0, SL, wt, 0)

    load_sub(0, xba, sxa)

    def accum(j, _):
        ka = 2 * j
        kb = 2 * j + 1
        load_sub(kb, xbb, sxb)
        wait_sub(xba, sxa)
        pltpu.sync_copy(xba, acc.at[ids.at[pl.ds(ka * SUB, SUB)]], add=True)
        pltpu.async_copy(ones, cacc.at[ids.at[pl.ds(ka * SUB, SUB)]],
                         semc, add=True)
        kn = jnp.minimum(ka + 2, NSUB - 1)
        load_sub(kn, xba, sxa)
        wait_sub(xbb, sxb)
        pltpu.sync_copy(xbb, acc.at[ids.at[pl.ds(kb * SUB, SUB)]], add=True)
        pltpu.async_copy(ones, cacc.at[ids.at[pl.ds(kb * SUB, SUB)]],
                         semc, add=True)
        return 0
    lax.fori_loop(0, NSUB // 2, accum, 0)
    wait_sub(xba, sxa)

    def drain_counts(k, _):
        pltpu.make_async_copy(ones, cacc.at[ids.at[pl.ds(0, SUB)]],
                              semc).wait()
        return 0
    lax.fori_loop(0, NSUB, drain_counts, 0)

    trt = base_rt + NSUB * SL

    @pl.when(s < NT - 1)
    def _():
        def ld(m, _):
            pltpu.async_copy(x_hbm.at[trt + m, c],
                             xba.at[pl.ds(SL * m, SL)], sxa)
            return 0
        lax.fori_loop(0, TAIL // SL, ld, 0)

        def wt(m, _):
            pltpu.make_async_copy(x_hbm.at[trt, c],
                                  xba.at[pl.ds(SL * m, SL)], sxa).wait()
            return 0
        lax.fori_loop(0, TAIL // SL, wt, 0)
        isl = ids.at[pl.ds(NSUB * SUB, TAIL)]
        pltpu.sync_copy(xba.at[pl.ds(0, TAIL)], acc.at[isl], add=True)
        pltpu.sync_copy(ones.at[pl.ds(0, TAIL)], cacc.at[isl], add=True)

    @pl.when(s == NT - 1)
    def _():
        pltpu.sync_copy(x_hbm.at[trt, c], xba.at[pl.ds(0, TAIL_L)])
        isl = ids.at[pl.ds(NSUB * SUB, TAIL_L)]
        pltpu.sync_copy(xba.at[pl.ds(0, TAIL_L)], acc.at[isl], add=True)
        pltpu.sync_copy(ones.at[pl.ds(0, TAIL_L)], cacc.at[isl], add=True)

    plsc.subcore_barrier()

    sl = pl.ds(s * CPT, CPT)
    pltpu.sync_copy(acc.at[sl], sums_hbm.at[c, sl])

    @pl.when(c == 0)
    def _():
        pltpu.sync_copy(cacc.at[sl], cnts_hbm.at[sl])


def _tc_divide(sums_ref, cnts_ref, out_ref):
    cnt = cnts_ref[:, 0:1].astype(jnp.int32) & 0xFFFF
    inv = 1.0 / jnp.maximum(cnt.astype(jnp.float32), 1.0)
    out_ref[:, :HD] = sums_ref[0] * inv
    out_ref[:, HD:] = sums_ref[1] * inv


@jax.jit
def _pooled(x4, cm):
    mesh = plsc.VectorSubcoreMesh(core_axis_name="c", subcore_axis_name="s")
    f = functools.partial(
        pl.kernel,
        mesh=mesh,
        out_type=(
            jax.ShapeDtypeStruct((NSC, CPAD, HD), jnp.float32),
            jax.ShapeDtypeStruct((CPAD, CW), jnp.int16),
        ),
        scratch_types=[
            pltpu.VMEM_SHARED((CPAD, HD), jnp.float32),
            pltpu.VMEM_SHARED((CPAD, CW), jnp.int16),
            pltpu.VMEM((RPT,), jnp.int32),
            pltpu.VMEM((SUB, HD), jnp.float32),
            pltpu.VMEM((SUB, HD), jnp.float32),
            pltpu.VMEM((SUB, CW), jnp.int16),
            pltpu.SemaphoreType.DMA,
            pltpu.SemaphoreType.DMA,
            pltpu.SemaphoreType.DMA,
        ],
        compiler_params=pltpu.CompilerParams(
            use_tc_tiling_on_sc=False, needs_layout_passes=False
        ),
    )(_sc_body)
    sums, cnts = f(x4, cm)

    blk = 8 * CPT
    out = pl.pallas_call(
        _tc_divide,
        grid=(CPAD // blk,),
        in_specs=[
            pl.BlockSpec((NSC, blk, HD), lambda i: (0, i, 0)),
            pl.BlockSpec((blk, CW), lambda i: (i, 0)),
        ],
        out_specs=pl.BlockSpec((blk, D_FEAT), lambda i: (i, 0)),
        out_shape=jax.ShapeDtypeStruct((NUM_CLUSTERS, D_FEAT), jnp.float32),
    )(sums, cnts)
    return out


def kernel(x, cluster_map, edge_index):
    x4 = x.reshape(NRT, SL, NSC, HD).transpose(0, 2, 1, 3)
    return _pooled(x4, cluster_map), edge_index
